# Initial kernel scaffold; baseline (speedup 1.0000x reference)
import jax
import jax.numpy as jnp
from jax.experimental import pallas as pl


def kernel(x, edge_index, W_in, b_in, g_in, be_in, Wl0, bl0, Wr0, g0, be0, Wl1, bl1, Wr1, g1, be1, Wl2, bl2, Wr2, g2, be2, W_out, b_out):
    raise NotImplementedError("write your pallas kernel here")


# speedup vs baseline: 2.6328x; 2.6328x over previous
import functools

import jax
import jax.numpy as jnp
from jax import lax
from jax.experimental import pallas as pl
from jax.experimental.pallas import tpu as pltpu
from jax.experimental.pallas import tpu_sc as plsc

N = 10000
E = 320000
H = 128
C = 64
EPS = 1e-5

NC = 2
NS = 16
NW = NC * NS
CHUNK = 128
EPW = 10240
NCH = EPW // CHUNK
GRP = 8
NGRP = NCH // GRP
ROWS = 10112
RPT = ROWS // NS
DUMMY = N

_f32 = jnp.float32


def _sc_aggregate(h, srcp, dstp, zeros_h):
  mesh = plsc.VectorSubcoreMesh(core_axis_name="c", subcore_axis_name="s")

  def body(h_hbm, srcp_hbm, dstp_hbm, z_hbm, agg_hbm,
           sidx, didx, rows_a, rows_b, agg_s, sem_a, sem_b):
    c = lax.axis_index("c")
    s = lax.axis_index("s")
    wid = c * NS + s

    pltpu.sync_copy(z_hbm, agg_s.at[pl.ds(s * RPT, RPT)])
    plsc.subcore_barrier()

    def group(gi, carry):
      g0 = pl.multiple_of(gi * GRP, GRP)
      pltpu.sync_copy(srcp_hbm.at[wid, pl.ds(g0, GRP)], sidx)
      pltpu.sync_copy(dstp_hbm.at[wid, pl.ds(g0, GRP)], didx)
      for j in range(0, GRP, 2):
        cp0 = pltpu.async_copy(h_hbm.at[sidx.at[j]], rows_a, sem_a)
        cp1 = pltpu.async_copy(h_hbm.at[sidx.at[j + 1]], rows_b, sem_b)
        cp0.wait()
        pltpu.sync_copy(rows_a, agg_s.at[didx.at[j]], add=True)
        cp1.wait()
        pltpu.sync_copy(rows_b, agg_s.at[didx.at[j + 1]], add=True)
      return carry

    lax.fori_loop(0, NGRP, group, 0)
    plsc.subcore_barrier()

    pltpu.sync_copy(agg_s.at[pl.ds(s * RPT, RPT)],
                    agg_hbm.at[c, pl.ds(s * RPT, RPT)])

  fn = pl.kernel(
      body,
      out_type=jax.ShapeDtypeStruct((NC, ROWS, H), _f32),
      mesh=mesh,
      scratch_types=(
          pltpu.VMEM((GRP, CHUNK), jnp.int32),
          pltpu.VMEM((GRP, CHUNK), jnp.int32),
          pltpu.VMEM((CHUNK, H), _f32),
          pltpu.VMEM((CHUNK, H), _f32),
          pltpu.VMEM_SHARED((ROWS, H), _f32),
          pltpu.SemaphoreType.DMA,
          pltpu.SemaphoreType.DMA,
      ),
  )
  return fn(h, srcp, dstp, zeros_h)


def _sc_degree(dstp, zeros_h, ones_d):
  mesh = plsc.VectorSubcoreMesh(core_axis_name="c", subcore_axis_name="s")

  def body(dstp_hbm, zd_hbm, od_hbm, deg_hbm, dst_v, ones_v, deg_s):
    c = lax.axis_index("c")
    s = lax.axis_index("s")
    wid = c * NS + s

    pltpu.sync_copy(dstp_hbm.at[wid], dst_v)
    pltpu.sync_copy(od_hbm, ones_v)
    pltpu.sync_copy(zd_hbm, deg_s.at[pl.ds(s * RPT, RPT)])
    plsc.subcore_barrier()

    def step(i, carry):
      pltpu.sync_copy(ones_v, deg_s.at[dst_v.at[i]], add=True)
      return carry

    lax.fori_loop(0, NCH, step, 0)
    plsc.subcore_barrier()
    pltpu.sync_copy(deg_s.at[pl.ds(s * RPT, RPT)],
                    deg_hbm.at[c, pl.ds(s * RPT, RPT)])

  fn = pl.kernel(
      body,
      out_type=jax.ShapeDtypeStruct((NC, ROWS, H), _f32),
      mesh=mesh,
      scratch_types=(
          pltpu.VMEM((NCH, CHUNK), jnp.int32),
          pltpu.VMEM((CHUNK, H), _f32),
          pltpu.VMEM_SHARED((ROWS, H), _f32),
      ),
  )
  return fn(dstp, zeros_h, ones_d)


def _bn_relu(z, g, be):
  m = jnp.mean(z, axis=0, keepdims=True)
  zc = z - m
  v = jnp.mean(zc * zc, axis=0, keepdims=True)
  zn = zc * lax.rsqrt(v + EPS) * g + be
  return jnp.maximum(zn, 0.0)


def _tc_input(x, wt, b, g, be):
  def body(x_ref, wt_ref, b_ref, g_ref, be_ref, o_ref):
    z = jnp.dot(x_ref[...], wt_ref[...], preferred_element_type=_f32)
    o_ref[...] = _bn_relu(z + b_ref[...], g_ref[...], be_ref[...])

  return pl.pallas_call(
      body, out_shape=jax.ShapeDtypeStruct((N, H), _f32))(
          x, wt, b.reshape(1, H), g.reshape(1, H), be.reshape(1, H))


def _tc_layer(h, agg, deg, wlt, bl, wrt, g, be, wot=None, bo=None):
  final = wot is not None

  def body(*refs):
    if final:
      (h_ref, agg_ref, deg_ref, wlt_ref, bl_ref, wrt_ref, g_ref, be_ref,
       wot_ref, bo_ref, o_ref) = refs
    else:
      (h_ref, agg_ref, deg_ref, wlt_ref, bl_ref, wrt_ref, g_ref, be_ref,
       o_ref) = refs
    hv = h_ref[...]
    a = agg_ref[...]
    asum = a[0, :N, :] + a[1, :N, :]
    d = deg_ref[...]
    dsum = d[0, :N, 0:1] + d[1, :N, 0:1]
    scale = 1.0 / jnp.maximum(dsum, 1.0)
    t = (jnp.dot(asum * scale, wlt_ref[...], preferred_element_type=_f32)
         + bl_ref[...]
         + jnp.dot(hv, wrt_ref[...], preferred_element_type=_f32))
    hn = hv + _bn_relu(t, g_ref[...], be_ref[...])
    if final:
      o_ref[...] = (jnp.dot(hn, wot_ref[...], preferred_element_type=_f32)
                    + bo_ref[...])
    else:
      o_ref[...] = hn

  args = [h, agg, deg, wlt, bl.reshape(1, H), wrt, g.reshape(1, H),
          be.reshape(1, H)]
  if final:
    args += [wot, bo.reshape(1, C)]
    out = jax.ShapeDtypeStruct((N, C), _f32)
  else:
    out = jax.ShapeDtypeStruct((N, H), _f32)
  return pl.pallas_call(body, out_shape=out)(*args)


def kernel(x, edge_index, W_in, b_in, g_in, be_in,
           Wl0, bl0, Wr0, g0, be0,
           Wl1, bl1, Wr1, g1, be1,
           Wl2, bl2, Wr2, g2, be2,
           W_out, b_out):
  src = edge_index[0]
  dst = edge_index[1]
  pad = NW * EPW - E
  srcp = jnp.concatenate([src, jnp.zeros((pad,), jnp.int32)]
                         ).reshape(NW, NCH, CHUNK)
  dstp = jnp.concatenate([dst, jnp.full((pad,), DUMMY, jnp.int32)]
                         ).reshape(NW, NCH, CHUNK)
  zeros_h = jnp.zeros((RPT, H), _f32)
  ones_d = jnp.ones((CHUNK, H), _f32)

  h = _tc_input(x, W_in.T, b_in, g_in, be_in)
  deg = _sc_degree(dstp, zeros_h, ones_d)[:, :, :16]

  agg0 = _sc_aggregate(h, srcp, dstp, zeros_h)
  h = _tc_layer(h, agg0, deg, Wl0.T, bl0, Wr0.T, g0, be0)

  agg1 = _sc_aggregate(h, srcp, dstp, zeros_h)
  h = _tc_layer(h, agg1, deg, Wl1.T, bl1, Wr1.T, g1, be1)

  agg2 = _sc_aggregate(h, srcp, dstp, zeros_h)
  return _tc_layer(h, agg2, deg, Wl2.T, bl2, Wr2.T, g2, be2,
                   wot=W_out.T, bo=b_out)


# speedup vs baseline: 2.6372x; 1.0017x over previous
import functools

import jax
import jax.numpy as jnp
from jax import lax
from jax.experimental import pallas as pl
from jax.experimental.pallas import tpu as pltpu
from jax.experimental.pallas import tpu_sc as plsc

N = 10000
E = 320000
H = 128
C = 64
EPS = 1e-5

NC = 2
NS = 16
NW = NC * NS
CHUNK = 128
EPW = 10240
NCH = EPW // CHUNK
GRP = 8
NGRP = NCH // GRP
ROWS = 10112
RPT = ROWS // NS
DUMMY = N

_f32 = jnp.float32


def _sc_aggregate(h, srcp, dstp, zeros_h):
  mesh = plsc.VectorSubcoreMesh(core_axis_name="c", subcore_axis_name="s")

  def body(h_hbm, srcp_hbm, dstp_hbm, z_hbm, agg_hbm,
           sidx, didx, rows_a, rows_b, agg_s, sem_a, sem_b):
    c = lax.axis_index("c")
    s = lax.axis_index("s")
    wid = c * NS + s

    pltpu.sync_copy(z_hbm, agg_s.at[pl.ds(s * RPT, RPT)])
    plsc.subcore_barrier()

    def group(gi, carry):
      g0 = pl.multiple_of(gi * GRP, GRP)
      pltpu.sync_copy(srcp_hbm.at[wid, pl.ds(g0, GRP)], sidx)
      pltpu.sync_copy(dstp_hbm.at[wid, pl.ds(g0, GRP)], didx)
      for j in range(0, GRP, 2):
        cp0 = pltpu.async_copy(h_hbm.at[sidx.at[j]], rows_a, sem_a)
        cp1 = pltpu.async_copy(h_hbm.at[sidx.at[j + 1]], rows_b, sem_b)
        cp0.wait()
        pltpu.sync_copy(rows_a, agg_s.at[didx.at[j]], add=True)
        cp1.wait()
        pltpu.sync_copy(rows_b, agg_s.at[didx.at[j + 1]], add=True)
      return carry

    lax.fori_loop(0, NGRP, group, 0)
    plsc.subcore_barrier()

    pltpu.sync_copy(agg_s.at[pl.ds(s * RPT, RPT)],
                    agg_hbm.at[c, pl.ds(s * RPT, RPT)])

  fn = pl.kernel(
      body,
      out_type=jax.ShapeDtypeStruct((NC, ROWS, H), _f32),
      mesh=mesh,
      scratch_types=(
          pltpu.VMEM((GRP, CHUNK), jnp.int32),
          pltpu.VMEM((GRP, CHUNK), jnp.int32),
          pltpu.VMEM((CHUNK, H), _f32),
          pltpu.VMEM((CHUNK, H), _f32),
          pltpu.VMEM_SHARED((ROWS, H), _f32),
          pltpu.SemaphoreType.DMA,
          pltpu.SemaphoreType.DMA,
      ),
  )
  return fn(h, srcp, dstp, zeros_h)


def _sc_degree(dstp, zeros_h, ones_d):
  mesh = plsc.VectorSubcoreMesh(core_axis_name="c", subcore_axis_name="s")

  def body(dstp_hbm, zd_hbm, od_hbm, deg_hbm, dst_v, ones_v, deg_s):
    c = lax.axis_index("c")
    s = lax.axis_index("s")
    wid = c * NS + s

    pltpu.sync_copy(dstp_hbm.at[wid], dst_v)
    pltpu.sync_copy(od_hbm, ones_v)
    pltpu.sync_copy(zd_hbm, deg_s.at[pl.ds(s * RPT, RPT)])
    plsc.subcore_barrier()

    def step(i, carry):
      pltpu.sync_copy(ones_v, deg_s.at[dst_v.at[i]], add=True)
      return carry

    lax.fori_loop(0, NCH, step, 0)
    plsc.subcore_barrier()
    pltpu.sync_copy(deg_s.at[pl.ds(s * RPT, RPT)],
                    deg_hbm.at[c, pl.ds(s * RPT, RPT)])

  fn = pl.kernel(
      body,
      out_type=jax.ShapeDtypeStruct((NC, ROWS, H), _f32),
      mesh=mesh,
      scratch_types=(
          pltpu.VMEM((NCH, CHUNK), jnp.int32),
          pltpu.VMEM((CHUNK, H), _f32),
          pltpu.VMEM_SHARED((ROWS, H), _f32),
      ),
  )
  return fn(dstp, zeros_h, ones_d)


def _bn_relu(z, g, be):
  m = jnp.mean(z, axis=0, keepdims=True)
  zc = z - m
  v = jnp.mean(zc * zc, axis=0, keepdims=True)
  zn = zc * lax.rsqrt(v + EPS) * g + be
  return jnp.maximum(zn, 0.0)


def _tc_input(x, wt, b, g, be):
  def body(x_ref, wt_ref, b_ref, g_ref, be_ref, o_ref):
    z = jnp.dot(x_ref[...], wt_ref[...], preferred_element_type=_f32)
    o_ref[...] = _bn_relu(z + b_ref[...], g_ref[...], be_ref[...])

  return pl.pallas_call(
      body, out_shape=jax.ShapeDtypeStruct((N, H), _f32))(
          x, wt, b.reshape(1, H), g.reshape(1, H), be.reshape(1, H))


def _tc_layer(h, agg, deg, wlt, bl, wrt, g, be, wot=None, bo=None):
  final = wot is not None

  def body(*refs):
    if final:
      (h_ref, agg_ref, deg_ref, wlt_ref, bl_ref, wrt_ref, g_ref, be_ref,
       wot_ref, bo_ref, o_ref) = refs
    else:
      (h_ref, agg_ref, deg_ref, wlt_ref, bl_ref, wrt_ref, g_ref, be_ref,
       o_ref) = refs
    hv = h_ref[...]
    a = agg_ref[...]
    asum = a[0, :N, :] + a[1, :N, :]
    d = deg_ref[...]
    dsum = d[0, :N, 0:1] + d[1, :N, 0:1]
    scale = 1.0 / jnp.maximum(dsum, 1.0)
    t = (jnp.dot(asum * scale, wlt_ref[...], preferred_element_type=_f32)
         + bl_ref[...]
         + jnp.dot(hv, wrt_ref[...], preferred_element_type=_f32))
    hn = hv + _bn_relu(t, g_ref[...], be_ref[...])
    if final:
      o_ref[...] = (jnp.dot(hn, wot_ref[...], preferred_element_type=_f32)
                    + bo_ref[...])
    else:
      o_ref[...] = hn

  args = [h, agg, deg, wlt, bl.reshape(1, H), wrt, g.reshape(1, H),
          be.reshape(1, H)]
  if final:
    args += [wot, bo.reshape(1, C)]
    out = jax.ShapeDtypeStruct((N, C), _f32)
  else:
    out = jax.ShapeDtypeStruct((N, H), _f32)
  return pl.pallas_call(body, out_shape=out)(*args)


def kernel(x, edge_index, W_in, b_in, g_in, be_in,
           Wl0, bl0, Wr0, g0, be0,
           Wl1, bl1, Wr1, g1, be1,
           Wl2, bl2, Wr2, g2, be2,
           W_out, b_out):
  src = edge_index[0]
  dst = edge_index[1]
  pad = NW * EPW - E
  srcp = jnp.concatenate([src, jnp.zeros((pad,), jnp.int32)]
                         ).reshape(NW, NCH, CHUNK)
  pad_dst = DUMMY + jnp.arange(pad, dtype=jnp.int32) % (ROWS - N)
  dstp = jnp.concatenate([dst, pad_dst]).reshape(NW, NCH, CHUNK)
  zeros_h = jnp.zeros((RPT, H), _f32)
  ones_d = jnp.ones((CHUNK, H), _f32)

  h = _tc_input(x, W_in.T, b_in, g_in, be_in)
  deg = _sc_degree(dstp, zeros_h, ones_d)[:, :, :16]

  agg0 = _sc_aggregate(h, srcp, dstp, zeros_h)
  h = _tc_layer(h, agg0, deg, Wl0.T, bl0, Wr0.T, g0, be0)

  agg1 = _sc_aggregate(h, srcp, dstp, zeros_h)
  h = _tc_layer(h, agg1, deg, Wl1.T, bl1, Wr1.T, g1, be1)

  agg2 = _sc_aggregate(h, srcp, dstp, zeros_h)
  return _tc_layer(h, agg2, deg, Wl2.T, bl2, Wr2.T, g2, be2,
                   wot=W_out.T, bo=b_out)


# speedup vs baseline: 2.9903x; 1.1339x over previous
import functools

import jax
import jax.numpy as jnp
from jax import lax
from jax.experimental import pallas as pl
from jax.experimental.pallas import tpu as pltpu
from jax.experimental.pallas import tpu_sc as plsc

N = 10000
E = 320000
H = 128
C = 64
EPS = 1e-5

NC = 2
NS = 16
NW = NC * NS
CHUNK = 80
EPW = 10240
NCH = EPW // CHUNK
GRP = 8
NGRP = NCH // GRP
NBUF = 4
ROWS = 10112
RPT = ROWS // NS
DUMMY = N

_f32 = jnp.float32


def _sc_aggregate(h, srcp, dstp, zeros_h):
  mesh = plsc.VectorSubcoreMesh(core_axis_name="c", subcore_axis_name="s")

  def body(h_hbm, srcp_hbm, dstp_hbm, z_hbm, agg_hbm,
           sidx, didx, b0, b1, b2, b3, agg_s, *sems):
    bufs = (b0, b1, b2, b3)
    gsem = sems[:NBUF]
    ssem = sems[NBUF:]
    c = lax.axis_index("c")
    s = lax.axis_index("s")
    wid = c * NS + s

    pltpu.sync_copy(z_hbm, agg_s.at[pl.ds(s * RPT, RPT)])
    plsc.subcore_barrier()

    def group(gi, carry):
      g0 = pl.multiple_of(gi * GRP, GRP)
      pltpu.sync_copy(srcp_hbm.at[wid, pl.ds(g0, GRP)], sidx)
      pltpu.sync_copy(dstp_hbm.at[wid, pl.ds(g0, GRP)], didx)
      gat = [None] * GRP
      scat = [None] * GRP
      for j in range(GRP):
        b = j % NBUF
        if j >= NBUF:
          scat[j - NBUF].wait()
        gat[j] = pltpu.async_copy(h_hbm.at[sidx.at[j]], bufs[b], gsem[b])
        if j >= 1:
          bp = (j - 1) % NBUF
          gat[j - 1].wait()
          scat[j - 1] = pltpu.async_copy(
              bufs[bp], agg_s.at[didx.at[j - 1]], ssem[bp], add=True)
      gat[GRP - 1].wait()
      bp = (GRP - 1) % NBUF
      scat[GRP - 1] = pltpu.async_copy(
          bufs[bp], agg_s.at[didx.at[GRP - 1]], ssem[bp], add=True)
      for j in range(GRP - NBUF, GRP):
        scat[j].wait()
      return carry

    lax.fori_loop(0, NGRP, group, 0)
    plsc.subcore_barrier()

    pltpu.sync_copy(agg_s.at[pl.ds(s * RPT, RPT)],
                    agg_hbm.at[c, pl.ds(s * RPT, RPT)])

  fn = pl.kernel(
      body,
      out_type=jax.ShapeDtypeStruct((NC, ROWS, H), _f32),
      mesh=mesh,
      scratch_types=(
          pltpu.VMEM((GRP, CHUNK), jnp.int32),
          pltpu.VMEM((GRP, CHUNK), jnp.int32),
          pltpu.VMEM((CHUNK, H), _f32),
          pltpu.VMEM((CHUNK, H), _f32),
          pltpu.VMEM((CHUNK, H), _f32),
          pltpu.VMEM((CHUNK, H), _f32),
          pltpu.VMEM_SHARED((ROWS, H), _f32),
          pltpu.SemaphoreType.DMA,
          pltpu.SemaphoreType.DMA,
          pltpu.SemaphoreType.DMA,
          pltpu.SemaphoreType.DMA,
          pltpu.SemaphoreType.DMA,
          pltpu.SemaphoreType.DMA,
          pltpu.SemaphoreType.DMA,
          pltpu.SemaphoreType.DMA,
      ),
  )
  return fn(h, srcp, dstp, zeros_h)


def _sc_degree(dstp, zeros_h, ones_d):
  mesh = plsc.VectorSubcoreMesh(core_axis_name="c", subcore_axis_name="s")

  def body(dstp_hbm, zd_hbm, od_hbm, deg_hbm, dst_v, ones_v, deg_s):
    c = lax.axis_index("c")
    s = lax.axis_index("s")
    wid = c * NS + s

    pltpu.sync_copy(dstp_hbm.at[wid], dst_v)
    pltpu.sync_copy(od_hbm, ones_v)
    pltpu.sync_copy(zd_hbm, deg_s.at[pl.ds(s * RPT, RPT)])
    plsc.subcore_barrier()

    def step(i, carry):
      pltpu.sync_copy(ones_v, deg_s.at[dst_v.at[i]], add=True)
      return carry

    lax.fori_loop(0, NCH, step, 0)
    plsc.subcore_barrier()
    pltpu.sync_copy(deg_s.at[pl.ds(s * RPT, RPT)],
                    deg_hbm.at[c, pl.ds(s * RPT, RPT)])

  fn = pl.kernel(
      body,
      out_type=jax.ShapeDtypeStruct((NC, ROWS, H), _f32),
      mesh=mesh,
      scratch_types=(
          pltpu.VMEM((NCH, CHUNK), jnp.int32),
          pltpu.VMEM((CHUNK, H), _f32),
          pltpu.VMEM_SHARED((ROWS, H), _f32),
      ),
  )
  return fn(dstp, zeros_h, ones_d)


def _bn_relu(z, g, be):
  m = jnp.mean(z, axis=0, keepdims=True)
  zc = z - m
  v = jnp.mean(zc * zc, axis=0, keepdims=True)
  zn = zc * lax.rsqrt(v + EPS) * g + be
  return jnp.maximum(zn, 0.0)


def _tc_input(x, wt, b, g, be):
  def body(x_ref, wt_ref, b_ref, g_ref, be_ref, o_ref):
    z = jnp.dot(x_ref[...], wt_ref[...], preferred_element_type=_f32)
    o_ref[...] = _bn_relu(z + b_ref[...], g_ref[...], be_ref[...])

  return pl.pallas_call(
      body, out_shape=jax.ShapeDtypeStruct((N, H), _f32))(
          x, wt, b.reshape(1, H), g.reshape(1, H), be.reshape(1, H))


def _tc_layer(h, agg, deg, wlt, bl, wrt, g, be, wot=None, bo=None):
  final = wot is not None

  def body(*refs):
    if final:
      (h_ref, agg_ref, deg_ref, wlt_ref, bl_ref, wrt_ref, g_ref, be_ref,
       wot_ref, bo_ref, o_ref) = refs
    else:
      (h_ref, agg_ref, deg_ref, wlt_ref, bl_ref, wrt_ref, g_ref, be_ref,
       o_ref) = refs
    hv = h_ref[...]
    a = agg_ref[...]
    asum = a[0, :N, :] + a[1, :N, :]
    d = deg_ref[...]
    dsum = d[0, :N, 0:1] + d[1, :N, 0:1]
    scale = 1.0 / jnp.maximum(dsum, 1.0)
    t = (jnp.dot(asum * scale, wlt_ref[...], preferred_element_type=_f32)
         + bl_ref[...]
         + jnp.dot(hv, wrt_ref[...], preferred_element_type=_f32))
    hn = hv + _bn_relu(t, g_ref[...], be_ref[...])
    if final:
      o_ref[...] = (jnp.dot(hn, wot_ref[...], preferred_element_type=_f32)
                    + bo_ref[...])
    else:
      o_ref[...] = hn

  args = [h, agg, deg, wlt, bl.reshape(1, H), wrt, g.reshape(1, H),
          be.reshape(1, H)]
  if final:
    args += [wot, bo.reshape(1, C)]
    out = jax.ShapeDtypeStruct((N, C), _f32)
  else:
    out = jax.ShapeDtypeStruct((N, H), _f32)
  return pl.pallas_call(body, out_shape=out)(*args)


def kernel(x, edge_index, W_in, b_in, g_in, be_in,
           Wl0, bl0, Wr0, g0, be0,
           Wl1, bl1, Wr1, g1, be1,
           Wl2, bl2, Wr2, g2, be2,
           W_out, b_out):
  src = edge_index[0]
  dst = edge_index[1]
  pad = NW * EPW - E
  srcp = jnp.concatenate([src, jnp.zeros((pad,), jnp.int32)]
                         ).reshape(NW, NCH, CHUNK)
  pad_dst = DUMMY + jnp.arange(pad, dtype=jnp.int32) % (ROWS - N)
  dstp = jnp.concatenate([dst, pad_dst]).reshape(NW, NCH, CHUNK)
  zeros_h = jnp.zeros((RPT, H), _f32)
  ones_d = jnp.ones((CHUNK, H), _f32)

  h = _tc_input(x, W_in.T, b_in, g_in, be_in)
  deg = _sc_degree(dstp, zeros_h, ones_d)[:, :, :16]

  agg0 = _sc_aggregate(h, srcp, dstp, zeros_h)
  h = _tc_layer(h, agg0, deg, Wl0.T, bl0, Wr0.T, g0, be0)

  agg1 = _sc_aggregate(h, srcp, dstp, zeros_h)
  h = _tc_layer(h, agg1, deg, Wl1.T, bl1, Wr1.T, g1, be1)

  agg2 = _sc_aggregate(h, srcp, dstp, zeros_h)
  return _tc_layer(h, agg2, deg, Wl2.T, bl2, Wr2.T, g2, be2,
                   wot=W_out.T, bo=b_out)
